# baseline (speedup 1.0000x reference)
import functools

import jax
import jax.numpy as jnp
from jax import lax
from jax.experimental import pallas as pl
from jax.experimental.pallas import tpu as pltpu
from jax.experimental.pallas import tpu_sc as plsc

N = 10000
E = 320000
H = 128
NP = 10240
NC = 2
NS = 16
TRASH = 10200

CH = 128
NCHUNK = 160
EPT = NCHUNK * CH
SLICE = NP // NS
FH = H // NC
NBUF = 4
QR = NCHUNK // 4
QF = EPT // 4
RPC = NP // CH

_sc_mesh = functools.partial(
    plsc.VectorSubcoreMesh, core_axis_name="c", subcore_axis_name="s",
    num_cores=NC, num_subcores=NS)
_sc_params = pltpu.CompilerParams(needs_layout_passes=False,
                                  use_tc_tiling_on_sc=False)


def _zero_vec(ref, n):
    zero16 = jnp.zeros((16,), jnp.float32)

    def zbody(i, _):
        ref[pl.ds(i * 16, 16)] = zero16
        return ()
    lax.fori_loop(0, n // 16, zbody, (), unroll=8)


def _rsqrt16(x):
    i = plsc.bitcast(x, jnp.int32)
    i = 0x5F3759DF - lax.shift_right_arithmetic(i, 1)
    y = plsc.bitcast(i, jnp.float32)
    for _ in range(3):
        y = y * (1.5 - 0.5 * x * y * y)
    return y


def _sc_all(srcp2, dstp2, w1, b1, zrows):
    @functools.partial(
        pl.kernel,
        out_type=(jax.ShapeDtypeStruct((NP,), jnp.float32),
                  jax.ShapeDtypeStruct((NP, FH), jnp.float32),
                  jax.ShapeDtypeStruct((NP, FH), jnp.float32),
                  jax.ShapeDtypeStruct((NP, FH), jnp.float32),
                  jax.ShapeDtypeStruct((NP, FH), jnp.float32)),
        mesh=_sc_mesh(),
        compiler_params=_sc_params,
        scratch_types=[
            pltpu.VMEM((NP,), jnp.float32),
            pltpu.VMEM((NP,), jnp.float32),
            pltpu.VMEM((QR, CH), jnp.int32),
            pltpu.VMEM((QR, CH), jnp.int32),
            pltpu.VMEM((QR, CH), jnp.int32),
            pltpu.VMEM((QR, CH), jnp.int32),
            pltpu.VMEM((CH, FH), jnp.float32),
            pltpu.VMEM((CH, FH), jnp.float32),
            pltpu.VMEM((CH, FH), jnp.float32),
            pltpu.VMEM((CH, FH), jnp.float32),
            pltpu.VMEM((SLICE,), jnp.float32),
            pltpu.VMEM((SLICE,), jnp.float32),
            pltpu.VMEM((SLICE,), jnp.float32),
            pltpu.VMEM((SLICE,), jnp.float32),
            pltpu.VMEM((SLICE,), jnp.float32),
            pltpu.VMEM((SLICE,), jnp.float32),
            pltpu.VMEM((H,), jnp.float32),
            pltpu.VMEM((H,), jnp.float32),
            pltpu.VMEM_SHARED((NP,), jnp.float32),
            pltpu.VMEM_SHARED((NP,), jnp.float32),
            pltpu.VMEM_SHARED((NP,), jnp.float32),
            pltpu.VMEM_SHARED((NP,), jnp.float32),
            pltpu.VMEM_SHARED((NP, FH), jnp.float32),
        ] + [pltpu.SemaphoreType.DMA] * (2 * NBUF),
    )
    def k(srcp2_hbm, dstp2_hbm, w1_hbm, b1_hbm, z_hbm,
          ideg_hbm, t0_hbm, t1_hbm, o0_hbm, o1_hbm,
          iacc, oacc, src_q, dst_q, src2_q, dst2_q,
          gbuf0, gbuf1, gbuf2, gbuf3,
          red_v, ids, ods, nds, nss, a1s, w1_v, b1_v,
          ideg_sh, odeg_sh, a1_sh, f0_sh, acc,
          *sems):
        gbufs = (gbuf0, gbuf1, gbuf2, gbuf3)
        gsems = sems[:NBUF]
        ssems = sems[NBUF:]
        c = lax.axis_index("c")
        s = lax.axis_index("s")
        cbase = s * SLICE
        ones16 = jnp.ones((16,), jnp.float32)
        iota16 = lax.iota(jnp.int32, 16)

        pltpu.sync_copy(z_hbm, gbuf0)
        for zk in range(SLICE // CH):
            pltpu.sync_copy(gbuf0, acc.at[pl.ds(cbase + zk * CH, CH)])
        _zero_vec(red_v, SLICE)
        pltpu.sync_copy(red_v, ideg_sh.at[pl.ds(cbase, SLICE)])
        pltpu.sync_copy(red_v, odeg_sh.at[pl.ds(cbase, SLICE)])
        pltpu.sync_copy(red_v, a1_sh.at[pl.ds(cbase, SLICE)])
        pltpu.sync_copy(w1_hbm, w1_v)
        pltpu.sync_copy(b1_hbm, b1_v)
        plsc.subcore_barrier()

        _zero_vec(iacc, NP)
        _zero_vec(oacc, NP)
        for q in range(4):
            pltpu.sync_copy(
                srcp2_hbm.at[pl.ds(s * NCHUNK + q * QR, QR)], src_q)
            pltpu.sync_copy(
                dstp2_hbm.at[pl.ds(s * NCHUNK + q * QR, QR)], dst_q)

            def ebody(row, _):
                for kk in range(CH // 16):
                    sv = src_q[row, pl.ds(kk * 16, 16)]
                    dv = dst_q[row, pl.ds(kk * 16, 16)]
                    plsc.addupdate_scatter(oacc, [sv], ones16)
                    plsc.addupdate_scatter(iacc, [dv], ones16)
                return ()
            lax.fori_loop(0, QR, ebody, ())

        def build_iota(r):
            def rbody(row, _):
                for kk in range(CH // 16):
                    src2_q[row, pl.ds(kk * 16, 16)] = (
                        iota16 + (r * QR * CH + kk * 16)
                        + row * CH)
                return ()
            lax.fori_loop(0, QR, rbody, ())

        def fire_red(arr, sh, r, sem):
            def fbody(row, _):
                pltpu.async_copy(
                    arr.at[pl.ds((r * QR + row) * CH, CH)],
                    sh.at[src2_q.at[row]], sem, add=True)
                return ()
            lax.fori_loop(0, QR, fbody, ())

        def drain_red(arr, sh, r, sem):
            def dbody(row, _):
                pltpu.make_async_copy(
                    arr.at[pl.ds((r * QR + row) * CH, CH)],
                    sh.at[src2_q.at[row]], sem).wait()
                return ()
            lax.fori_loop(0, QR, dbody, ())

        for r in range(2):
            build_iota(r)
            fire_red(iacc, ideg_sh, r, gsems[0])
            fire_red(oacc, odeg_sh, r, gsems[1])
            drain_red(iacc, ideg_sh, r, gsems[0])
            drain_red(oacc, odeg_sh, r, gsems[1])
        plsc.subcore_barrier()

        pltpu.sync_copy(ideg_sh.at[pl.ds(cbase, SLICE)], ids)
        pltpu.sync_copy(odeg_sh.at[pl.ds(cbase, SLICE)], ods)

        def p2body(kk, _):
            id16 = ids[pl.ds(kk * 16, 16)]
            od16 = ods[pl.ds(kk * 16, 16)]
            ns16 = _rsqrt16(jnp.maximum(od16, 1.0))
            nds[pl.ds(kk * 16, 16)] = _rsqrt16(jnp.maximum(id16, 1.0))
            nss[pl.ds(kk * 16, 16)] = ns16
            red_v[pl.ds(kk * 16, 16)] = id16 * ns16
            return ()
        lax.fori_loop(0, SLICE // 16, p2body, (), unroll=4)
        pltpu.sync_copy(red_v, f0_sh.at[pl.ds(cbase, SLICE)])

        @pl.when(c == 0)
        def _():
            pltpu.sync_copy(ids, ideg_hbm.at[pl.ds(cbase, SLICE)])
        plsc.subcore_barrier()

        f0_v = oacc
        pltpu.sync_copy(f0_sh, f0_v)
        a1acc = iacc
        _zero_vec(a1acc, NP)
        for q in range(4):
            pltpu.sync_copy(
                srcp2_hbm.at[pl.ds(s * NCHUNK + q * QR, QR)], src_q)
            pltpu.sync_copy(
                dstp2_hbm.at[pl.ds(s * NCHUNK + q * QR, QR)], dst_q)

            def abody(row, _):
                for kk in range(CH // 16):
                    sv = src_q[row, pl.ds(kk * 16, 16)]
                    dv = dst_q[row, pl.ds(kk * 16, 16)]
                    vals = plsc.load_gather(f0_v, [sv])
                    plsc.addupdate_scatter(a1acc, [dv], vals)
                return ()
            lax.fori_loop(0, QR, abody, ())

        for r in range(2):
            build_iota(r)
            fire_red(a1acc, a1_sh, r, gsems[0])
            drain_red(a1acc, a1_sh, r, gsems[0])
        plsc.subcore_barrier()

        pltpu.sync_copy(a1_sh.at[pl.ds(cbase, SLICE)], a1s)

        def sbody(kk, _):
            a1s[pl.ds(kk * 16, 16)] = (a1s[pl.ds(kk * 16, 16)]
                                       * nds[pl.ds(kk * 16, 16)])
            return ()
        lax.fori_loop(0, SLICE // 16, sbody, (), unroll=4)

        wbase = c * FH
        for t in range(SLICE // CH):
            def tbody(nl, _):
                nn = t * CH + nl
                bidx = jnp.full((16,), nn, jnp.int32)
                a1n = plsc.load_gather(a1s, [bidx])
                nsn = plsc.load_gather(nss, [bidx])
                for f in range(FH // 16):
                    w116 = w1_v[pl.ds(wbase + f * 16, 16)]
                    b116 = b1_v[pl.ds(wbase + f * 16, 16)]
                    vec = jnp.maximum(a1n * w116 + b116, 0.0) * nsn
                    gbuf0[nl, pl.ds(f * 16, 16)] = vec
                return ()
            lax.fori_loop(0, CH, tbody, ())

            @pl.when(c == 0)
            def _():
                pltpu.sync_copy(gbuf0,
                                t0_hbm.at[pl.ds(cbase + t * CH, CH)])

            @pl.when(c == 1)
            def _():
                pltpu.sync_copy(gbuf0,
                                t1_hbm.at[pl.ds(cbase + t * CH, CH)])
        plsc.subcore_barrier()

        def fire_gather(j, b):
            @pl.when(c == 0)
            def _():
                pltpu.async_copy(t0_hbm.at[src2_q.at[j]], gbufs[b],
                                 gsems[b])

            @pl.when(c == 1)
            def _():
                pltpu.async_copy(t1_hbm.at[src2_q.at[j]], gbufs[b],
                                 gsems[b])

        def wait_gather(j, b):
            @pl.when(c == 0)
            def _():
                pltpu.make_async_copy(t0_hbm.at[src2_q.at[j]], gbufs[b],
                                      gsems[b]).wait()

            @pl.when(c == 1)
            def _():
                pltpu.make_async_copy(t1_hbm.at[src2_q.at[j]], gbufs[b],
                                      gsems[b]).wait()

        def fire_scatter(j, b):
            pltpu.async_copy(gbufs[b], acc.at[dst2_q.at[j]], ssems[b],
                             add=True)

        def wait_scatter(j, b):
            pltpu.make_async_copy(gbufs[b], acc.at[dst2_q.at[j]],
                                  ssems[b]).wait()

        for q in range(4):
            pltpu.sync_copy(
                srcp2_hbm.at[pl.ds(s * NCHUNK + q * QR, QR)], src2_q)
            pltpu.sync_copy(
                dstp2_hbm.at[pl.ds(s * NCHUNK + q * QR, QR)], dst2_q)
            for b in range(NBUF):
                fire_gather(b, b)

            def body(jj, _):
                j = NBUF * jj
                for b in range(NBUF):
                    wait_gather(j + b, b)
                    fire_scatter(j + b, b)
                for b in range(NBUF):
                    wait_scatter(j + b, b)
                    fire_gather(j + NBUF + b, b)
                return ()
            lax.fori_loop(0, (QR - NBUF) // NBUF, body, ())

            j_last = QR - NBUF
            for b in range(NBUF):
                wait_gather(j_last + b, b)
                fire_scatter(j_last + b, b)
            for b in range(NBUF):
                wait_scatter(j_last + b, b)
        plsc.subcore_barrier()

        for zk in range(SLICE // CH):
            buf = gbufs[zk % NBUF]
            pltpu.sync_copy(acc.at[pl.ds(cbase + zk * CH, CH)], buf)

            @pl.when(c == 0)
            def _():
                pltpu.sync_copy(buf, o0_hbm.at[pl.ds(cbase + zk * CH, CH)])

            @pl.when(c == 1)
            def _():
                pltpu.sync_copy(buf, o1_hbm.at[pl.ds(cbase + zk * CH, CH)])

    return k(srcp2, dstp2, w1, b1, zrows)


def _tc_head_body(o0_ref, o1_ref, ideg_ref, w2_ref, b2_ref,
                  wc0_ref, bc0_ref, wc1_ref, bc1_ref, wc2_ref, bc2_ref,
                  wc3_ref, bc3_ref, wc4_ref, bc4_ref, out_ref):
    nd = lax.rsqrt(jnp.maximum(ideg_ref[...], 1.0))
    agg = jnp.concatenate([o0_ref[...], o1_ref[...]], axis=1) * nd
    h2 = jnp.dot(agg, w2_ref[...], preferred_element_type=jnp.float32)
    h2 = jnp.maximum(h2 + b2_ref[...], 0.0)
    rid = lax.broadcasted_iota(jnp.int32, (NP, 1), 0)
    h2 = jnp.where(rid < N, h2, 0.0)
    hg = jnp.sum(h2, axis=0, keepdims=True) * (1.0 / N)

    x = jnp.maximum(
        jnp.dot(hg, wc0_ref[...], preferred_element_type=jnp.float32)
        + bc0_ref[...], 0.0)
    x = jnp.maximum(
        jnp.dot(x, wc1_ref[...], preferred_element_type=jnp.float32)
        + bc1_ref[...], 0.0)
    x = jnp.maximum(
        jnp.dot(x, wc2_ref[...], preferred_element_type=jnp.float32)
        + bc2_ref[...], 0.0)
    x = jnp.maximum(
        jnp.dot(x, wc3_ref[...], preferred_element_type=jnp.float32)
        + bc3_ref[...], 0.0)
    logits = (jnp.dot(x, wc4_ref[...], preferred_element_type=jnp.float32)
              + bc4_ref[...])
    m = jnp.max(logits, axis=-1, keepdims=True)
    e = jnp.exp(logits - m)
    out_ref[...] = e / jnp.sum(e, axis=-1, keepdims=True)


def _tc_head(o0, o1, ideg, w2, b2r, wc0, bc0r, wc1, bc1r, wc2, bc2r,
             wc3, bc3r, wc4, bc4r):
    return pl.pallas_call(
        _tc_head_body,
        out_shape=jax.ShapeDtypeStruct((1, 10), jnp.float32),
    )(o0, o1, ideg, w2, b2r, wc0, bc0r, wc1, bc1r, wc2, bc2r, wc3, bc3r,
      wc4, bc4r)


def kernel(edge_index, W1, b1, W2, b2, Wc0, bc0, Wc1, bc1, Wc2, bc2,
           Wc3, bc3, Wc4, bc4):
    src = edge_index[0]
    dst = edge_index[1]

    pad = EPT - E // NS
    srcp2 = jnp.concatenate(
        [src.reshape(NS, E // NS),
         jnp.full((NS, pad), TRASH, jnp.int32)], axis=1).reshape(
             NS * NCHUNK, CH)
    dstp2 = jnp.concatenate(
        [dst.reshape(NS, E // NS),
         jnp.full((NS, pad), TRASH, jnp.int32)], axis=1).reshape(
             NS * NCHUNK, CH)
    zrows = jnp.zeros((CH, FH), jnp.float32)

    ideg, _t0, _t1, o0, o1 = _sc_all(srcp2, dstp2, W1.reshape(H), b1, zrows)
    return _tc_head(o0, o1, ideg.reshape(NP, 1), W2, b2.reshape(1, H),
                    Wc0, bc0.reshape(1, -1), Wc1, bc1.reshape(1, -1),
                    Wc2, bc2.reshape(1, -1), Wc3, bc3.reshape(1, -1),
                    Wc4, bc4.reshape(1, -1))


# speedup vs baseline: 1.0407x; 1.0407x over previous
import functools

import jax
import jax.numpy as jnp
from jax import lax
from jax.experimental import pallas as pl
from jax.experimental.pallas import tpu as pltpu
from jax.experimental.pallas import tpu_sc as plsc

N = 10000
E = 320000
H = 128
NP = 10240
NC = 2
NS = 16
NW = NC * NS
EP = E // NW
TRASH = 10200

CH = 128
NCHUNK = 160
EPT = NCHUNK * CH
ROWS_PER_TILE = NP // NS
FH = H // NC
NBUF = 5

_sc_mesh = functools.partial(
    plsc.VectorSubcoreMesh, core_axis_name="c", subcore_axis_name="s",
    num_cores=NC, num_subcores=NS)
_sc_params = pltpu.CompilerParams(needs_layout_passes=False,
                                  use_tc_tiling_on_sc=False)


SLICE = NP // NS


def _zero_vec(ref, n):
    zero16 = jnp.zeros((16,), jnp.float32)

    def zbody(i, _):
        ref[pl.ds(i * 16, 16)] = zero16
        return ()
    lax.fori_loop(0, n // 16, zbody, (), unroll=8)


def _reduce_core(s, acc, spart, tmp_v, red_v, out_hbm, out_base):
    pltpu.sync_copy(acc, spart.at[pl.ds(s * NP, NP)])
    plsc.subcore_barrier()
    cbase = s * SLICE
    pltpu.sync_copy(spart.at[pl.ds(cbase, SLICE)], red_v)

    def pbody(p, _):
        pltpu.sync_copy(spart.at[pl.ds(p * NP + cbase, SLICE)], tmp_v)

        def abody(k, _):
            red_v[pl.ds(k * 16, 16)] = (red_v[pl.ds(k * 16, 16)]
                                        + tmp_v[pl.ds(k * 16, 16)])
            return ()
        lax.fori_loop(0, SLICE // 16, abody, (), unroll=8)
        return ()
    lax.fori_loop(1, NS, pbody, ())
    pltpu.sync_copy(red_v, out_hbm.at[pl.ds(out_base + cbase, SLICE)])


EPC = E // NS


def _rsqrt16(x):
    i = plsc.bitcast(x, jnp.int32)
    i = 0x5F3759DF - lax.shift_right_arithmetic(i, 1)
    y = plsc.bitcast(i, jnp.float32)
    for _ in range(3):
        y = y * (1.5 - 0.5 * x * y * y)
    return y


def _sc_scalar(src, dst):
    @functools.partial(
        pl.kernel,
        out_type=(jax.ShapeDtypeStruct((NP,), jnp.float32),
                  jax.ShapeDtypeStruct((NP,), jnp.float32),
                  jax.ShapeDtypeStruct((NC * NP,), jnp.float32)),
        mesh=_sc_mesh(),
        compiler_params=_sc_params,
        scratch_types=[
            pltpu.VMEM((EPC,), jnp.int32),
            pltpu.VMEM((EPC,), jnp.int32),
            pltpu.VMEM((NP,), jnp.float32),
            pltpu.VMEM((NP,), jnp.float32),
            pltpu.VMEM((NP,), jnp.float32),
            pltpu.VMEM((SLICE,), jnp.float32),
            pltpu.VMEM((SLICE,), jnp.float32),
            pltpu.VMEM_SHARED((NS * NP,), jnp.float32),
            pltpu.VMEM_SHARED((NS * NP,), jnp.float32),
            pltpu.VMEM_SHARED((NP,), jnp.float32),
        ],
    )
    def k(src_hbm, dst_hbm, ideg_hbm, odeg_hbm, part_hbm,
          src_v, dst_v, iacc, oacc, f0_v, tmp_v, red_v, ispart, ospart,
          f0_sh):
        c = lax.axis_index("c")
        s = lax.axis_index("s")
        base = s * EPC
        pltpu.sync_copy(src_hbm.at[pl.ds(base, EPC)], src_v)
        pltpu.sync_copy(dst_hbm.at[pl.ds(base, EPC)], dst_v)

        _zero_vec(iacc, NP)
        _zero_vec(oacc, NP)

        ones16 = jnp.ones((16,), jnp.float32)

        def ebody(i, _):
            sv = src_v[pl.ds(i * 16, 16)]
            dv = dst_v[pl.ds(i * 16, 16)]
            plsc.addupdate_scatter(oacc, [sv], ones16)
            plsc.addupdate_scatter(iacc, [dv], ones16)
            return ()
        lax.fori_loop(0, EPC // 16, ebody, (), unroll=8)

        cbase = s * SLICE
        pltpu.sync_copy(iacc, ispart.at[pl.ds(s * NP, NP)])
        pltpu.sync_copy(oacc, ospart.at[pl.ds(s * NP, NP)])
        plsc.subcore_barrier()

        def red_slice(spart, out_v):
            pltpu.sync_copy(spart.at[pl.ds(cbase, SLICE)], out_v)

            def pbody(p, _):
                pltpu.sync_copy(spart.at[pl.ds(p * NP + cbase, SLICE)], tmp_v)

                def abody(kk, _):
                    out_v[pl.ds(kk * 16, 16)] = (out_v[pl.ds(kk * 16, 16)]
                                                 + tmp_v[pl.ds(kk * 16, 16)])
                    return ()
                lax.fori_loop(0, SLICE // 16, abody, (), unroll=8)
                return ()
            lax.fori_loop(1, NS, pbody, ())

        red_slice(ispart, red_v)

        odeg_slice = iacc
        pltpu.sync_copy(ospart.at[pl.ds(cbase, SLICE)],
                        odeg_slice.at[pl.ds(0, SLICE)])

        def pbody2(p, _):
            pltpu.sync_copy(ospart.at[pl.ds(p * NP + cbase, SLICE)], tmp_v)

            def abody(kk, _):
                odeg_slice[pl.ds(kk * 16, 16)] = (
                    odeg_slice[pl.ds(kk * 16, 16)] + tmp_v[pl.ds(kk * 16, 16)])
                return ()
            lax.fori_loop(0, SLICE // 16, abody, (), unroll=8)
            return ()
        lax.fori_loop(1, NS, pbody2, ())

        @pl.when(c == 0)
        def _():
            pltpu.sync_copy(red_v, ideg_hbm.at[pl.ds(cbase, SLICE)])

        @pl.when(c == 1)
        def _():
            pltpu.sync_copy(odeg_slice.at[pl.ds(0, SLICE)],
                            odeg_hbm.at[pl.ds(cbase, SLICE)])

        def fbody(kk, _):
            ideg16 = red_v[pl.ds(kk * 16, 16)]
            odeg16 = odeg_slice[pl.ds(kk * 16, 16)]
            f016 = ideg16 * _rsqrt16(jnp.maximum(odeg16, 1.0))
            tmp_v[pl.ds(kk * 16, 16)] = f016
            return ()
        lax.fori_loop(0, SLICE // 16, fbody, (), unroll=8)
        pltpu.sync_copy(tmp_v, f0_sh.at[pl.ds(cbase, SLICE)])
        plsc.subcore_barrier()

        pltpu.sync_copy(f0_sh, f0_v)
        acc = oacc
        _zero_vec(acc, NP)

        abase = c * EP

        def gbody(i, _):
            sv = src_v[pl.ds(abase + i * 16, 16)]
            dv = dst_v[pl.ds(abase + i * 16, 16)]
            vals = plsc.load_gather(f0_v, [sv])
            plsc.addupdate_scatter(acc, [dv], vals)
            return ()
        lax.fori_loop(0, EP // 16, gbody, (), unroll=8)

        _reduce_core(s, acc, ispart, tmp_v, red_v, part_hbm, c * NP)

    return k(src, dst)


def _sc_agg2(srcp, dstp, t0, t1, zrows):
    @functools.partial(
        pl.kernel,
        out_type=(jax.ShapeDtypeStruct((NP, FH), jnp.float32),
                  jax.ShapeDtypeStruct((NP, FH), jnp.float32)),
        mesh=_sc_mesh(),
        compiler_params=_sc_params,
        scratch_types=[
            pltpu.VMEM((NCHUNK, CH), jnp.int32),
            pltpu.VMEM((NCHUNK, CH), jnp.int32),
        ] + [pltpu.VMEM((CH, FH), jnp.float32)] * NBUF
          + [pltpu.VMEM_SHARED((NP, FH), jnp.float32)]
          + [pltpu.SemaphoreType.DMA] * (2 * NBUF),
    )
    def k(src_hbm, dst_hbm, t0_hbm, t1_hbm, z_hbm, o0_hbm, o1_hbm,
          src_v, dst_v, *rest):
        gbufs = rest[:NBUF]
        acc = rest[NBUF]
        gsems = rest[NBUF + 1:2 * NBUF + 1]
        ssems = rest[2 * NBUF + 1:]
        c = lax.axis_index("c")
        s = lax.axis_index("s")
        rbase = s * ROWS_PER_TILE

        pltpu.sync_copy(z_hbm, gbufs[0])
        for zk in range(ROWS_PER_TILE // CH):
            pltpu.sync_copy(gbufs[0], acc.at[pl.ds(rbase + zk * CH, CH)])

        pltpu.sync_copy(src_hbm.at[pl.ds(s * NCHUNK, NCHUNK)], src_v)
        pltpu.sync_copy(dst_hbm.at[pl.ds(s * NCHUNK, NCHUNK)], dst_v)
        plsc.subcore_barrier()

        def fire_gather(j, b):
            @pl.when(c == 0)
            def _():
                pltpu.async_copy(t0_hbm.at[src_v.at[j]], gbufs[b], gsems[b])

            @pl.when(c == 1)
            def _():
                pltpu.async_copy(t1_hbm.at[src_v.at[j]], gbufs[b], gsems[b])

        def wait_gather(j, b):
            @pl.when(c == 0)
            def _():
                pltpu.make_async_copy(
                    t0_hbm.at[src_v.at[j]], gbufs[b], gsems[b]).wait()

            @pl.when(c == 1)
            def _():
                pltpu.make_async_copy(
                    t1_hbm.at[src_v.at[j]], gbufs[b], gsems[b]).wait()

        def fire_scatter(j, b):
            pltpu.async_copy(gbufs[b], acc.at[dst_v.at[j]], ssems[b],
                             add=True)

        def wait_scatter(j, b):
            pltpu.make_async_copy(gbufs[b], acc.at[dst_v.at[j]],
                                  ssems[b]).wait()

        for b in range(NBUF):
            fire_gather(b, b)

        def body(jj, _):
            j = NBUF * jj
            for b in range(NBUF):
                wait_gather(j + b, b)
                fire_scatter(j + b, b)
            for b in range(NBUF):
                wait_scatter(j + b, b)
                fire_gather(j + NBUF + b, b)
            return ()
        lax.fori_loop(0, (NCHUNK - NBUF) // NBUF, body, ())

        j_last = NCHUNK - NBUF
        for b in range(NBUF):
            wait_gather(j_last + b, b)
            fire_scatter(j_last + b, b)
        for b in range(NBUF):
            wait_scatter(j_last + b, b)

        plsc.subcore_barrier()

        for zk in range(ROWS_PER_TILE // CH):
            buf = gbufs[zk % NBUF]
            pltpu.sync_copy(acc.at[pl.ds(rbase + zk * CH, CH)], buf)

            @pl.when(c == 0)
            def _():
                pltpu.sync_copy(buf, o0_hbm.at[pl.ds(rbase + zk * CH, CH)])

            @pl.when(c == 1)
            def _():
                pltpu.sync_copy(buf, o1_hbm.at[pl.ds(rbase + zk * CH, CH)])

    return k(srcp, dstp, t0, t1, zrows)


def _tc_h1_body(a1p_ref, ideg_ref, odeg_ref, w1_ref, b1_ref,
                t0_ref, t1_ref, nd_ref):
    nd = lax.rsqrt(jnp.maximum(ideg_ref[...], 1.0))
    ns_ = lax.rsqrt(jnp.maximum(odeg_ref[...], 1.0))
    a1 = jnp.sum(a1p_ref[...], axis=0) * nd
    hs = jnp.maximum(a1 * w1_ref[...] + b1_ref[...], 0.0) * ns_
    t0_ref[...] = hs[:, :FH]
    t1_ref[...] = hs[:, FH:]
    nd_ref[...] = nd


def _tc_h1(a1parts, ideg, odeg, w1, b1r):
    return pl.pallas_call(
        _tc_h1_body,
        out_shape=(jax.ShapeDtypeStruct((NP, FH), jnp.float32),
                   jax.ShapeDtypeStruct((NP, FH), jnp.float32),
                   jax.ShapeDtypeStruct((NP, 1), jnp.float32)),
    )(a1parts, ideg, odeg, w1, b1r)


def _tc_head_body(o0_ref, o1_ref, nd_ref, w2_ref, b2_ref,
                  wc0_ref, bc0_ref, wc1_ref, bc1_ref, wc2_ref, bc2_ref,
                  wc3_ref, bc3_ref, wc4_ref, bc4_ref, out_ref):
    agg = jnp.concatenate([o0_ref[...], o1_ref[...]], axis=1) * nd_ref[...]
    h2 = jnp.dot(agg, w2_ref[...], preferred_element_type=jnp.float32)
    h2 = jnp.maximum(h2 + b2_ref[...], 0.0)
    rid = lax.broadcasted_iota(jnp.int32, (NP, 1), 0)
    h2 = jnp.where(rid < N, h2, 0.0)
    hg = jnp.sum(h2, axis=0, keepdims=True) * (1.0 / N)

    x = jnp.maximum(
        jnp.dot(hg, wc0_ref[...], preferred_element_type=jnp.float32)
        + bc0_ref[...], 0.0)
    x = jnp.maximum(
        jnp.dot(x, wc1_ref[...], preferred_element_type=jnp.float32)
        + bc1_ref[...], 0.0)
    x = jnp.maximum(
        jnp.dot(x, wc2_ref[...], preferred_element_type=jnp.float32)
        + bc2_ref[...], 0.0)
    x = jnp.maximum(
        jnp.dot(x, wc3_ref[...], preferred_element_type=jnp.float32)
        + bc3_ref[...], 0.0)
    logits = (jnp.dot(x, wc4_ref[...], preferred_element_type=jnp.float32)
              + bc4_ref[...])
    m = jnp.max(logits, axis=-1, keepdims=True)
    e = jnp.exp(logits - m)
    out_ref[...] = e / jnp.sum(e, axis=-1, keepdims=True)


def _tc_head(o0, o1, nd, w2, b2r, wc0, bc0r, wc1, bc1r, wc2, bc2r,
             wc3, bc3r, wc4, bc4r):
    return pl.pallas_call(
        _tc_head_body,
        out_shape=jax.ShapeDtypeStruct((1, 10), jnp.float32),
    )(o0, o1, nd, w2, b2r, wc0, bc0r, wc1, bc1r, wc2, bc2r, wc3, bc3r,
      wc4, bc4r)


def kernel(edge_index, W1, b1, W2, b2, Wc0, bc0, Wc1, bc1, Wc2, bc2,
           Wc3, bc3, Wc4, bc4):
    src = edge_index[0]
    dst = edge_index[1]

    pad = EPT - E // NS
    srcp = jnp.concatenate(
        [src.reshape(NS, E // NS),
         jnp.zeros((NS, pad), jnp.int32)], axis=1).reshape(NS * NCHUNK, CH)
    dstp = jnp.concatenate(
        [dst.reshape(NS, E // NS),
         jnp.full((NS, pad), TRASH, jnp.int32)], axis=1).reshape(
             NS * NCHUNK, CH)
    zrows = jnp.zeros((CH, FH), jnp.float32)

    ideg, odeg, a1parts = _sc_scalar(src, dst)
    t0, t1, nd = _tc_h1(a1parts.reshape(NC, NP, 1), ideg.reshape(NP, 1),
                        odeg.reshape(NP, 1), W1, b1.reshape(1, H))
    o0, o1 = _sc_agg2(srcp, dstp, t0, t1, zrows)
    return _tc_head(o0, o1, nd, W2, b2.reshape(1, H),
                    Wc0, bc0.reshape(1, -1), Wc1, bc1.reshape(1, -1),
                    Wc2, bc2.reshape(1, -1), Wc3, bc3.reshape(1, -1),
                    Wc4, bc4.reshape(1, -1))
